# Initial kernel scaffold; baseline (speedup 1.0000x reference)
import jax
import jax.numpy as jnp
from jax.experimental import pallas as pl


def kernel(x, adj, A, W_enc_pr, b_enc_pr, W_msg_pr, W_upd_pr, w_node_pr, We1_pr, We2_pr, W_enc_bfs, b_enc_bfs, W_msg_bfs, W_upd_bfs, w_node_bfs, We1_bfs, We2_bfs, phase_logits, lengths):
    raise NotImplementedError("write your pallas kernel here")


# speedup vs baseline: 14.3654x; 14.3654x over previous
import functools
import math

import jax
import jax.numpy as jnp
from jax.experimental import pallas as pl
from jax.experimental.pallas import tpu as pltpu

B, N, F, H, T = 8, 512, 128, 128, 16
MASKED = -1.0
_INV_SQRT_H = 1.0 / math.sqrt(H)


def _edge_kernel(ns_ref, skip_ref, x_ref, adj_ref, a_ref,
                 we_pr, be_pr, wm_pr, wu_pr,
                 we_bf, be_bf, wm_bf, wu_bf, we1_bf, we2_bf,
                 out_ref):
    b = pl.program_id(0)
    ns = ns_ref[b]
    sk = skip_ref[b]

    @pl.when(sk == 0)
    def _compute():
        x = x_ref[0]
        adj = adj_ref[0]
        f32 = jnp.float32

        z_pr = jnp.tanh(jnp.dot(x, we_pr[...], preferred_element_type=f32)
                        + be_pr[...])
        zm_pr = jnp.dot(z_pr, wm_pr[...], preferred_element_type=f32)
        zu_pr = jnp.dot(z_pr, wu_pr[0:H, :], preferred_element_type=f32)
        wu_pr_lo = wu_pr[H:2 * H, :]

        def pr_step(h):
            m = jnp.maximum(zm_pr + jnp.dot(h, wm_pr[...],
                                            preferred_element_type=f32), 0.0)
            msg = jnp.dot(adj, m, preferred_element_type=f32)
            return jnp.maximum(zu_pr + jnp.dot(msg, wu_pr_lo,
                                               preferred_element_type=f32), 0.0)

        m0 = jnp.maximum(zm_pr, 0.0)
        msg0 = jnp.dot(adj, m0, preferred_element_type=f32)
        h0 = jnp.maximum(zu_pr + jnp.dot(msg0, wu_pr_lo,
                                         preferred_element_type=f32), 0.0)
        h = jax.lax.fori_loop(1, ns, lambda i, hh: pr_step(hh), h0)

        z_bf = jnp.tanh(jnp.dot(x, we_bf[...], preferred_element_type=f32)
                        + be_bf[...])
        m2 = jnp.maximum(jnp.dot(z_bf + h, wm_bf[...],
                                 preferred_element_type=f32), 0.0)
        msg2 = jnp.dot(adj, m2, preferred_element_type=f32)
        hb = jnp.maximum(jnp.dot(z_bf, wu_bf[0:H, :], preferred_element_type=f32)
                         + jnp.dot(msg2, wu_bf[H:2 * H, :],
                                   preferred_element_type=f32), 0.0)
        e1 = jnp.dot(hb, we1_bf[...], preferred_element_type=f32)
        e2 = jnp.dot(hb, we2_bf[...], preferred_element_type=f32)
        cand = jax.lax.dot_general(e1, e2, (((1,), (1,)), ((), ())),
                                   preferred_element_type=f32) * _INV_SQRT_H
        out_ref[0] = a_ref[0] * cand

    @pl.when(sk != 0)
    def _masked():
        out_ref[0] = jnp.full((N, N), MASKED, jnp.float32)


def kernel(x, adj, A, W_enc_pr, b_enc_pr, W_msg_pr, W_upd_pr, w_node_pr,
           We1_pr, We2_pr, W_enc_bfs, b_enc_bfs, W_msg_bfs, W_upd_bfs,
           w_node_bfs, We1_bfs, We2_bfs, phase_logits, lengths):
    del w_node_pr, We1_pr, We2_pr, w_node_bfs

    p = jnp.argmax(phase_logits, axis=-1).astype(jnp.int32)
    iv = jnp.arange(T, dtype=jnp.int32)[:, None]
    ln = lengths.astype(jnp.int32)[None, :]
    valid = (iv >= 1) & (ln > iv + 1) & (p == 0)
    i_star = jnp.max(jnp.where(valid, iv, 0), axis=0)
    reset = (p == 1) & (iv < i_star[None, :])
    j0 = jnp.max(jnp.where(reset, iv + 1, 0), axis=0)
    nsteps = i_star - j0 + 1
    skip = ((i_star == 0) & (p[0] == 1)).astype(jnp.int32)

    be_pr = b_enc_pr.reshape(1, H)
    be_bf = b_enc_bfs.reshape(1, H)

    smem = pl.BlockSpec(memory_space=pltpu.SMEM)
    full = lambda *shape: pl.BlockSpec(shape, lambda b: (0,) * len(shape))
    batched = lambda *shape: pl.BlockSpec((1,) + shape, lambda b: (b, 0, 0))

    out = pl.pallas_call(
        _edge_kernel,
        grid=(B,),
        in_specs=[
            smem, smem,
            batched(N, F), batched(N, N), batched(N, N),
            full(F, H), full(1, H), full(H, H), full(2 * H, H),
            full(F, H), full(1, H), full(H, H), full(2 * H, H),
            full(H, H), full(H, H),
        ],
        out_specs=batched(N, N),
        out_shape=jax.ShapeDtypeStruct((B, N, N), jnp.float32),
    )(nsteps, skip, x, adj, A,
      W_enc_pr, be_pr, W_msg_pr, W_upd_pr,
      W_enc_bfs, be_bf, W_msg_bfs, W_upd_bfs, We1_bfs, We2_bfs)
    return out
